# baseline (speedup 1.0000x reference)
import functools

import jax
import jax.numpy as jnp
from jax import lax
from jax.experimental import pallas as pl
from jax.experimental.pallas import tpu as pltpu
from jax.experimental.pallas import tpu_sc as plsc

NC, NS, L = 2, 16, 16
NW = NC * NS
CH = 80
NB = 3
NBI = 4
FR = 128
TC_R = 1000


def _pad_rows(n):
    rpt = -(-(-(-n // NS)) // FR) * FR
    return rpt, NS * rpt


def _fill_f32(ref, nrows, ncols16, value):
    v = jnp.full((L,), value, jnp.float32)

    def row(i, _):
        def col(j, __):
            ref[i, pl.ds(j * L, L)] = v
            return 0
        return lax.fori_loop(0, ncols16, col, 0)

    lax.fori_loop(0, nrows, row, 0)


def _mesh():
    return plsc.VectorSubcoreMesh(core_axis_name="c", subcore_axis_name="s")


_SC_PARAMS = pltpu.CompilerParams(use_tc_tiling_on_sc=False)


@functools.lru_cache(maxsize=None)
def _deg_kernel(n_nodes, e_pad):
    nit = e_pad // (NW * CH)
    rpt, n_pad = _pad_rows(n_nodes)

    @functools.partial(
        pl.kernel,
        out_type=jax.ShapeDtypeStruct((NC, n_pad, L), jnp.float32),
        mesh=_mesh(),
        scratch_types=[
            pltpu.VMEM((NBI, CH), jnp.int32),
            pltpu.VMEM((CH, L), jnp.float32),
            pltpu.VMEM_SHARED((n_pad, L), jnp.float32),
            pltpu.SemaphoreType.DMA((NBI,)),
            pltpu.SemaphoreType.DMA((NBI,)),
        ],
        compiler_params=_SC_PARAMS,
    )
    def deg(srcf_hbm, dacc_hbm, iring, ones_v, dacc_sh, isem, ssem):
        c = lax.axis_index("c")
        s = lax.axis_index("s")
        wid = c * NS + s

        _fill_f32(ones_v, CH, 1, 0.0)

        def zloop(j, _):
            pltpu.sync_copy(ones_v, dacc_sh.at[pl.ds(s * rpt + j * CH, CH)])
            return 0
        lax.fori_loop(0, rpt // CH, zloop, 0)
        _fill_f32(ones_v, CH, 1, 1.0)
        plsc.subcore_barrier()

        def istart(j):
            b = lax.rem(j, NBI)
            pltpu.async_copy(srcf_hbm.at[pl.ds((wid * nit + j) * CH, CH)],
                             iring.at[b], isem.at[b])

        def iwait(j):
            b = lax.rem(j, NBI)
            pltpu.make_async_copy(srcf_hbm.at[pl.ds((wid * nit + j) * CH, CH)],
                                  iring.at[b], isem.at[b]).wait()

        def sstart(i):
            b = lax.rem(i, NBI)
            pltpu.async_copy(ones_v, dacc_sh.at[iring.at[b]], ssem.at[b],
                             add=True)

        def swait(i):
            b = lax.rem(i, NBI)
            pltpu.make_async_copy(ones_v, dacc_sh.at[iring.at[b]],
                                  ssem.at[b]).wait()

        istart(0)
        istart(1)

        def chunk(i, _):
            jj = i + 2
            @pl.when(jj < nit)
            def _():
                @pl.when(jj >= NBI)
                def _():
                    swait(jj - NBI)
                istart(jj)
            iwait(i)
            sstart(i)
            return 0
        lax.fori_loop(0, nit, chunk, 0)

        def drain(t, _):
            swait(nit - NBI + t)
            return 0
        lax.fori_loop(0, NBI, drain, 0)
        plsc.subcore_barrier()

        def flush(j, _):
            r0 = s * rpt + j * FR
            pltpu.sync_copy(dacc_sh.at[pl.ds(r0, FR)], dacc_hbm.at[c, pl.ds(r0, FR)])
            return 0
        lax.fori_loop(0, rpt // FR, flush, 0)

    return deg


@functools.lru_cache(maxsize=None)
def _spmm_kernel(n_nodes, e_pad, d):
    nit = e_pad // (NW * CH)
    rpt, n_pad = _pad_rows(n_nodes)

    @functools.partial(
        pl.kernel,
        out_type=jax.ShapeDtypeStruct((NC, n_pad, d), jnp.float32),
        mesh=_mesh(),
        scratch_types=[
            pltpu.VMEM((NBI, CH), jnp.int32),
            pltpu.VMEM((NBI, CH), jnp.int32),
            pltpu.VMEM((NB, CH, d), jnp.float32),
            pltpu.VMEM_SHARED((n_pad, d), jnp.float32),
            pltpu.SemaphoreType.DMA((NBI,)),
            pltpu.SemaphoreType.DMA((NB,)),
            pltpu.SemaphoreType.DMA((NB,)),
        ],
        compiler_params=_SC_PARAMS,
    )
    def spmm(hw_hbm, srcf_hbm, dstf_hbm, sout_hbm, isr_v, idr_v, rows_v,
             s_sh, isem, gsem, ssem):
        c = lax.axis_index("c")
        s = lax.axis_index("s")
        wid = c * NS + s

        def zfill(i, _):
            def col(j, __):
                rows_v[0, i, pl.ds(j * L, L)] = jnp.zeros((L,), jnp.float32)
                return 0
            return lax.fori_loop(0, d // L, col, 0)
        lax.fori_loop(0, CH, zfill, 0)

        def zloop(j, _):
            pltpu.sync_copy(rows_v.at[0], s_sh.at[pl.ds(s * rpt + j * CH, CH)])
            return 0
        lax.fori_loop(0, rpt // CH, zloop, 0)
        plsc.subcore_barrier()

        def istart(j):
            b = lax.rem(j, NBI)
            off = (wid * nit + j) * CH
            pltpu.async_copy(srcf_hbm.at[pl.ds(off, CH)], isr_v.at[b], isem.at[b])
            pltpu.async_copy(dstf_hbm.at[pl.ds(off, CH)], idr_v.at[b], isem.at[b])

        def iwait(j):
            b = lax.rem(j, NBI)
            off = (wid * nit + j) * CH
            pltpu.make_async_copy(
                srcf_hbm.at[pl.ds(off, CH)], isr_v.at[b], isem.at[b]).wait()
            pltpu.make_async_copy(
                dstf_hbm.at[pl.ds(off, CH)], idr_v.at[b], isem.at[b]).wait()

        def gstart(j):
            pltpu.async_copy(hw_hbm.at[isr_v.at[lax.rem(j, NBI)]],
                             rows_v.at[lax.rem(j, NB)], gsem.at[lax.rem(j, NB)])

        def gwait(j):
            pltpu.make_async_copy(hw_hbm.at[isr_v.at[lax.rem(j, NBI)]],
                                  rows_v.at[lax.rem(j, NB)],
                                  gsem.at[lax.rem(j, NB)]).wait()

        def sstart(i):
            pltpu.async_copy(rows_v.at[lax.rem(i, NB)],
                             s_sh.at[idr_v.at[lax.rem(i, NBI)]],
                             ssem.at[lax.rem(i, NB)], add=True)

        def swait(i):
            pltpu.make_async_copy(rows_v.at[lax.rem(i, NB)],
                                  s_sh.at[idr_v.at[lax.rem(i, NBI)]],
                                  ssem.at[lax.rem(i, NB)]).wait()

        istart(0)
        istart(1)
        iwait(0)
        gstart(0)

        def chunk(i, _):
            j2 = i + 2
            @pl.when(j2 < nit)
            def _():
                istart(j2)
            j1 = i + 1
            @pl.when(j1 < nit)
            def _():
                @pl.when(j1 >= NB)
                def _():
                    swait(j1 - NB)
                iwait(j1)
                gstart(j1)
            gwait(i)
            sstart(i)
            return 0
        lax.fori_loop(0, nit, chunk, 0)

        def drain(t, _):
            swait(nit - NB + t)
            return 0
        lax.fori_loop(0, NB, drain, 0)
        plsc.subcore_barrier()

        def flush(j, _):
            r0 = s * rpt + j * FR
            pltpu.sync_copy(s_sh.at[pl.ds(r0, FR)], sout_hbm.at[c, pl.ds(r0, FR)])
            return 0
        lax.fori_loop(0, rpt // FR, flush, 0)

    return spmm


def _dinv(dacc_blk):
    deg = dacc_blk[0, :, 0:1] + dacc_blk[1, :, 0:1] + 1.0
    return lax.rsqrt(deg)


@functools.lru_cache(maxsize=None)
def _tc_mm(n, d):
    def body(x_ref, w_ref, b_ref, out_ref):
        hw = jnp.dot(x_ref[...], w_ref[...], preferred_element_type=jnp.float32)
        out_ref[...] = hw + b_ref[...]

    return pl.pallas_call(
        body,
        grid=(n // TC_R,),
        in_specs=[
            pl.BlockSpec((TC_R, d), lambda i: (i, 0)),
            pl.BlockSpec((d, d), lambda i: (0, 0)),
            pl.BlockSpec((1, d), lambda i: (0, 0)),
        ],
        out_specs=pl.BlockSpec((TC_R, d), lambda i: (i, 0)),
        out_shape=jax.ShapeDtypeStruct((n, d), jnp.float32),
    )


@functools.lru_cache(maxsize=None)
def _tc_scale(n, d):
    def body(hw_ref, dacc_ref, out_ref):
        out_ref[...] = hw_ref[...] * _dinv(dacc_ref[...])

    return pl.pallas_call(
        body,
        grid=(n // TC_R,),
        in_specs=[
            pl.BlockSpec((TC_R, d), lambda i: (i, 0)),
            pl.BlockSpec((NC, TC_R, L), lambda i: (0, i, 0)),
        ],
        out_specs=pl.BlockSpec((TC_R, d), lambda i: (i, 0)),
        out_shape=jax.ShapeDtypeStruct((n, d), jnp.float32),
    )


@functools.lru_cache(maxsize=None)
def _tc_mid(n, d):
    def body(dacc_ref, s_ref, hwp_ref, w_ref, b_ref, out_ref):
        dinv = _dinv(dacc_ref[...])
        sm = s_ref[0] + s_ref[1] + hwp_ref[...]
        h = jnp.maximum(dinv * sm, 0.0)
        hw = jnp.dot(h, w_ref[...], preferred_element_type=jnp.float32)
        out_ref[...] = (hw + b_ref[...]) * dinv

    return pl.pallas_call(
        body,
        grid=(n // TC_R,),
        in_specs=[
            pl.BlockSpec((NC, TC_R, L), lambda i: (0, i, 0)),
            pl.BlockSpec((NC, TC_R, d), lambda i: (0, i, 0)),
            pl.BlockSpec((TC_R, d), lambda i: (i, 0)),
            pl.BlockSpec((d, d), lambda i: (0, 0)),
            pl.BlockSpec((1, d), lambda i: (0, 0)),
        ],
        out_specs=pl.BlockSpec((TC_R, d), lambda i: (i, 0)),
        out_shape=jax.ShapeDtypeStruct((n, d), jnp.float32),
    )


@functools.lru_cache(maxsize=None)
def _tc_last(n, d):
    def body(dacc_ref, s_ref, hwp_ref, out_ref):
        dinv = _dinv(dacc_ref[...])
        sm = s_ref[0] + s_ref[1] + hwp_ref[...]
        out_ref[...] = jnp.maximum(dinv * sm, 0.0)

    return pl.pallas_call(
        body,
        grid=(n // TC_R,),
        in_specs=[
            pl.BlockSpec((NC, TC_R, L), lambda i: (0, i, 0)),
            pl.BlockSpec((NC, TC_R, d), lambda i: (0, i, 0)),
            pl.BlockSpec((TC_R, d), lambda i: (i, 0)),
        ],
        out_specs=pl.BlockSpec((TC_R, d), lambda i: (i, 0)),
        out_shape=jax.ShapeDtypeStruct((n, d), jnp.float32),
    )


def kernel(edge_index, X, W1, b1, W2, b2):
    n, d = X.shape
    e = edge_index.shape[1]
    assert n % TC_R == 0 and d % L == 0

    _, n_pad = _pad_rows(n)
    block = NW * CH
    nit = max(8, -(-e // block))
    e_pad = NW * nit * CH
    npadlen = e_pad - e

    b1r = b1.reshape(1, d)
    b2r = b2.reshape(1, d)
    src = edge_index[0]
    dst = edge_index[1]
    if npadlen:
        pad_last = jnp.full((npadlen,), n_pad - 1, jnp.int32)
        src_deg = jnp.concatenate([src, pad_last])
        src_sp = jnp.concatenate([src, jnp.zeros((npadlen,), jnp.int32)])
        dst_sp = jnp.concatenate([dst, pad_last])
    else:
        src_deg = src_sp = src
        dst_sp = dst

    dacc = _deg_kernel(n, e_pad)(src_deg)
    hw1 = _tc_mm(n, d)(X, W1, b1r)
    hw1p = _tc_scale(n, d)(hw1, dacc)
    s1 = _spmm_kernel(n, e_pad, d)(hw1p, src_sp, dst_sp)
    hw2p = _tc_mid(n, d)(dacc, s1, hw1p, W2, b2r)
    s2 = _spmm_kernel(n, e_pad, d)(hw2p, src_sp, dst_sp)
    return _tc_last(n, d)(dacc, s2, hw2p)


# speedup vs baseline: 1.0226x; 1.0226x over previous
import functools

import jax
import jax.numpy as jnp
from jax import lax
from jax.experimental import pallas as pl
from jax.experimental.pallas import tpu as pltpu
from jax.experimental.pallas import tpu_sc as plsc

NC, NS, L = 2, 16, 16
NW = NC * NS
CH = 80
CHD = 128
NB = 3
NBI = 4
FR = 128
TC_R = 1000


def _pad_rows(n):
    rpt = -(-(-(-n // NS)) // FR) * FR
    return rpt, NS * rpt


def _fill_f32(ref, nrows, ncols16, value):
    v = jnp.full((L,), value, jnp.float32)

    def row(i, _):
        def col(j, __):
            ref[i, pl.ds(j * L, L)] = v
            return 0
        return lax.fori_loop(0, ncols16, col, 0)

    lax.fori_loop(0, nrows, row, 0)


def _mesh():
    return plsc.VectorSubcoreMesh(core_axis_name="c", subcore_axis_name="s")


_SC_PARAMS = pltpu.CompilerParams(use_tc_tiling_on_sc=False)


@functools.lru_cache(maxsize=None)
def _deg_kernel(n_nodes, e_pad):
    nit = e_pad // (NW * CHD)
    rpt, n_pad = _pad_rows(n_nodes)

    @functools.partial(
        pl.kernel,
        out_type=jax.ShapeDtypeStruct((NC, n_pad, L), jnp.float32),
        mesh=_mesh(),
        scratch_types=[
            pltpu.VMEM((NBI, CHD), jnp.int32),
            pltpu.VMEM((CHD, L), jnp.float32),
            pltpu.VMEM_SHARED((n_pad, L), jnp.float32),
            pltpu.SemaphoreType.DMA((NBI,)),
            pltpu.SemaphoreType.DMA((NBI,)),
        ],
        compiler_params=_SC_PARAMS,
    )
    def deg(srcf_hbm, dacc_hbm, iring, ones_v, dacc_sh, isem, ssem):
        c = lax.axis_index("c")
        s = lax.axis_index("s")
        wid = c * NS + s

        _fill_f32(ones_v, CHD, 1, 0.0)

        def zloop(j, _):
            pltpu.sync_copy(ones_v, dacc_sh.at[pl.ds(s * rpt + j * CHD, CHD)])
            return 0
        lax.fori_loop(0, rpt // CHD, zloop, 0)
        _fill_f32(ones_v, CHD, 1, 1.0)
        plsc.subcore_barrier()

        def istart(j):
            b = lax.rem(j, NBI)
            pltpu.async_copy(srcf_hbm.at[pl.ds((wid * nit + j) * CHD, CHD)],
                             iring.at[b], isem.at[b])

        def iwait(j):
            b = lax.rem(j, NBI)
            pltpu.make_async_copy(srcf_hbm.at[pl.ds((wid * nit + j) * CHD, CHD)],
                                  iring.at[b], isem.at[b]).wait()

        def sstart(i):
            b = lax.rem(i, NBI)
            pltpu.async_copy(ones_v, dacc_sh.at[iring.at[b]], ssem.at[b],
                             add=True)

        def swait(i):
            b = lax.rem(i, NBI)
            pltpu.make_async_copy(ones_v, dacc_sh.at[iring.at[b]],
                                  ssem.at[b]).wait()

        istart(0)
        istart(1)

        def chunk(i, _):
            jj = i + 2
            @pl.when(jj < nit)
            def _():
                @pl.when(jj >= NBI)
                def _():
                    swait(jj - NBI)
                istart(jj)
            iwait(i)
            sstart(i)
            return 0
        lax.fori_loop(0, nit, chunk, 0)

        def drain(t, _):
            swait(nit - NBI + t)
            return 0
        lax.fori_loop(0, NBI, drain, 0)
        plsc.subcore_barrier()

        def flush(j, _):
            r0 = s * rpt + j * FR
            pltpu.sync_copy(dacc_sh.at[pl.ds(r0, FR)], dacc_hbm.at[c, pl.ds(r0, FR)])
            return 0
        lax.fori_loop(0, rpt // FR, flush, 0)

    return deg


@functools.lru_cache(maxsize=None)
def _spmm_kernel(n_nodes, e_pad, d):
    nit = e_pad // (NW * CH)
    rpt, n_pad = _pad_rows(n_nodes)

    @functools.partial(
        pl.kernel,
        out_type=jax.ShapeDtypeStruct((NC, n_pad, d), jnp.float32),
        mesh=_mesh(),
        scratch_types=[
            pltpu.VMEM((NBI, CH), jnp.int32),
            pltpu.VMEM((NBI, CH), jnp.int32),
            pltpu.VMEM((NB, CH, d), jnp.float32),
            pltpu.VMEM_SHARED((n_pad, d), jnp.float32),
            pltpu.SemaphoreType.DMA((NBI,)),
            pltpu.SemaphoreType.DMA((NB,)),
            pltpu.SemaphoreType.DMA((NB,)),
        ],
        compiler_params=_SC_PARAMS,
    )
    def spmm(hw_hbm, srcf_hbm, dstf_hbm, sout_hbm, isr_v, idr_v, rows_v,
             s_sh, isem, gsem, ssem):
        c = lax.axis_index("c")
        s = lax.axis_index("s")
        wid = c * NS + s

        def istart(j):
            b = lax.rem(j, NBI)
            off = (wid * nit + j) * CH
            pltpu.async_copy(srcf_hbm.at[pl.ds(off, CH)], isr_v.at[b], isem.at[b])
            pltpu.async_copy(dstf_hbm.at[pl.ds(off, CH)], idr_v.at[b], isem.at[b])

        def iwait(j):
            b = lax.rem(j, NBI)
            off = (wid * nit + j) * CH
            pltpu.make_async_copy(
                srcf_hbm.at[pl.ds(off, CH)], isr_v.at[b], isem.at[b]).wait()
            pltpu.make_async_copy(
                dstf_hbm.at[pl.ds(off, CH)], idr_v.at[b], isem.at[b]).wait()

        def gstart(j):
            pltpu.async_copy(hw_hbm.at[isr_v.at[lax.rem(j, NBI)]],
                             rows_v.at[lax.rem(j, NB)], gsem.at[lax.rem(j, NB)])

        def gwait(j):
            pltpu.make_async_copy(hw_hbm.at[isr_v.at[lax.rem(j, NBI)]],
                                  rows_v.at[lax.rem(j, NB)],
                                  gsem.at[lax.rem(j, NB)]).wait()

        def sstart(i):
            pltpu.async_copy(rows_v.at[lax.rem(i, NB)],
                             s_sh.at[idr_v.at[lax.rem(i, NBI)]],
                             ssem.at[lax.rem(i, NB)], add=True)

        def swait(i):
            pltpu.make_async_copy(rows_v.at[lax.rem(i, NB)],
                                  s_sh.at[idr_v.at[lax.rem(i, NBI)]],
                                  ssem.at[lax.rem(i, NB)]).wait()

        istart(0)
        istart(1)
        iwait(0)
        gstart(0)

        def zfill(i, _):
            def col(j, __):
                rows_v[NB - 1, i, pl.ds(j * L, L)] = jnp.zeros((L,), jnp.float32)
                return 0
            return lax.fori_loop(0, d // L, col, 0)
        lax.fori_loop(0, CH, zfill, 0)

        def zloop(j, _):
            pltpu.sync_copy(rows_v.at[NB - 1], s_sh.at[pl.ds(s * rpt + j * CH, CH)])
            return 0
        lax.fori_loop(0, rpt // CH, zloop, 0)
        plsc.subcore_barrier()

        def chunk(i, _):
            j2 = i + 2
            @pl.when(j2 < nit)
            def _():
                istart(j2)
            j1 = i + 1
            @pl.when(j1 < nit)
            def _():
                @pl.when(j1 >= NB)
                def _():
                    swait(j1 - NB)
                iwait(j1)
                gstart(j1)
            gwait(i)
            sstart(i)
            return 0
        lax.fori_loop(0, nit, chunk, 0)

        def drain(t, _):
            swait(nit - NB + t)
            return 0
        lax.fori_loop(0, NB, drain, 0)
        plsc.subcore_barrier()

        def flush(j, _):
            r0 = s * rpt + j * FR
            pltpu.sync_copy(s_sh.at[pl.ds(r0, FR)], sout_hbm.at[c, pl.ds(r0, FR)])
            return 0
        lax.fori_loop(0, rpt // FR, flush, 0)

    return spmm


def _dinv(dacc_blk):
    deg = dacc_blk[0, :, 0:1] + dacc_blk[1, :, 0:1] + 1.0
    return lax.rsqrt(deg)


@functools.lru_cache(maxsize=None)
def _tc_mm(n, d):
    def body(x_ref, w_ref, b_ref, out_ref):
        hw = jnp.dot(x_ref[...], w_ref[...], preferred_element_type=jnp.float32)
        out_ref[...] = hw + b_ref[...]

    return pl.pallas_call(
        body,
        grid=(n // TC_R,),
        in_specs=[
            pl.BlockSpec((TC_R, d), lambda i: (i, 0)),
            pl.BlockSpec((d, d), lambda i: (0, 0)),
            pl.BlockSpec((1, d), lambda i: (0, 0)),
        ],
        out_specs=pl.BlockSpec((TC_R, d), lambda i: (i, 0)),
        out_shape=jax.ShapeDtypeStruct((n, d), jnp.float32),
    )


@functools.lru_cache(maxsize=None)
def _tc_scale(n, d):
    def body(hw_ref, dacc_ref, out_ref):
        out_ref[...] = hw_ref[...] * _dinv(dacc_ref[...])

    return pl.pallas_call(
        body,
        grid=(n // TC_R,),
        in_specs=[
            pl.BlockSpec((TC_R, d), lambda i: (i, 0)),
            pl.BlockSpec((NC, TC_R, L), lambda i: (0, i, 0)),
        ],
        out_specs=pl.BlockSpec((TC_R, d), lambda i: (i, 0)),
        out_shape=jax.ShapeDtypeStruct((n, d), jnp.float32),
    )


@functools.lru_cache(maxsize=None)
def _tc_mid(n, d):
    def body(dacc_ref, s_ref, hwp_ref, w_ref, b_ref, out_ref):
        dinv = _dinv(dacc_ref[...])
        sm = s_ref[0] + s_ref[1] + hwp_ref[...]
        h = jnp.maximum(dinv * sm, 0.0)
        hw = jnp.dot(h, w_ref[...], preferred_element_type=jnp.float32)
        out_ref[...] = (hw + b_ref[...]) * dinv

    return pl.pallas_call(
        body,
        grid=(n // TC_R,),
        in_specs=[
            pl.BlockSpec((NC, TC_R, L), lambda i: (0, i, 0)),
            pl.BlockSpec((NC, TC_R, d), lambda i: (0, i, 0)),
            pl.BlockSpec((TC_R, d), lambda i: (i, 0)),
            pl.BlockSpec((d, d), lambda i: (0, 0)),
            pl.BlockSpec((1, d), lambda i: (0, 0)),
        ],
        out_specs=pl.BlockSpec((TC_R, d), lambda i: (i, 0)),
        out_shape=jax.ShapeDtypeStruct((n, d), jnp.float32),
    )


@functools.lru_cache(maxsize=None)
def _tc_last(n, d):
    def body(dacc_ref, s_ref, hwp_ref, out_ref):
        dinv = _dinv(dacc_ref[...])
        sm = s_ref[0] + s_ref[1] + hwp_ref[...]
        out_ref[...] = jnp.maximum(dinv * sm, 0.0)

    return pl.pallas_call(
        body,
        grid=(n // TC_R,),
        in_specs=[
            pl.BlockSpec((NC, TC_R, L), lambda i: (0, i, 0)),
            pl.BlockSpec((NC, TC_R, d), lambda i: (0, i, 0)),
            pl.BlockSpec((TC_R, d), lambda i: (i, 0)),
        ],
        out_specs=pl.BlockSpec((TC_R, d), lambda i: (i, 0)),
        out_shape=jax.ShapeDtypeStruct((n, d), jnp.float32),
    )


def kernel(edge_index, X, W1, b1, W2, b2):
    n, d = X.shape
    e = edge_index.shape[1]
    assert n % TC_R == 0 and d % L == 0

    _, n_pad = _pad_rows(n)
    block = NW * CH
    nit = max(8, -(-e // block))
    e_pad = NW * nit * CH
    npadlen = e_pad - e
    blockd = NW * CHD
    nitd = max(8, -(-e // blockd))
    e_pad_d = NW * nitd * CHD
    npadd = e_pad_d - e

    b1r = b1.reshape(1, d)
    b2r = b2.reshape(1, d)
    src = edge_index[0]
    dst = edge_index[1]
    if npadd:
        src_deg = jnp.concatenate(
            [src, jnp.full((npadd,), n_pad - 1, jnp.int32)])
    else:
        src_deg = src
    if npadlen:
        src_sp = jnp.concatenate([src, jnp.zeros((npadlen,), jnp.int32)])
        dst_sp = jnp.concatenate(
            [dst, jnp.full((npadlen,), n_pad - 1, jnp.int32)])
    else:
        src_sp = src
        dst_sp = dst

    dacc = _deg_kernel(n, e_pad_d)(src_deg)
    hw1 = _tc_mm(n, d)(X, W1, b1r)
    hw1p = _tc_scale(n, d)(hw1, dacc)
    s1 = _spmm_kernel(n, e_pad, d)(hw1p, src_sp, dst_sp)
    hw2p = _tc_mid(n, d)(dacc, s1, hw1p, W2, b2r)
    s2 = _spmm_kernel(n, e_pad, d)(hw2p, src_sp, dst_sp)
    return _tc_last(n, d)(dacc, s2, hw2p)


# speedup vs baseline: 1.0433x; 1.0202x over previous
import functools

import jax
import jax.numpy as jnp
from jax import lax
from jax.experimental import pallas as pl
from jax.experimental.pallas import tpu as pltpu
from jax.experimental.pallas import tpu_sc as plsc

NC, NS, L = 2, 16, 16
NW = NC * NS
CH = 80
NB = 3
NBI = 4
FR = 128
TC_R = 1000


def _pad_rows(n):
    rpt = -(-(-(-n // NS)) // FR) * FR
    return rpt, NS * rpt


def _fill_f32(ref, nrows, ncols16, value):
    v = jnp.full((L,), value, jnp.float32)

    def row(i, _):
        def col(j, __):
            ref[i, pl.ds(j * L, L)] = v
            return 0
        return lax.fori_loop(0, ncols16, col, 0)

    lax.fori_loop(0, nrows, row, 0)


def _mesh():
    return plsc.VectorSubcoreMesh(core_axis_name="c", subcore_axis_name="s")


_SC_PARAMS = pltpu.CompilerParams(use_tc_tiling_on_sc=False)


@functools.lru_cache(maxsize=None)
def _deg_kernel(n_nodes, nit, arrlen):
    rpt, n_pad = _pad_rows(n_nodes)

    @functools.partial(
        pl.kernel,
        out_type=jax.ShapeDtypeStruct((NC, n_pad, L), jnp.float32),
        mesh=_mesh(),
        scratch_types=[
            pltpu.VMEM((NBI, CH), jnp.int32),
            pltpu.VMEM((CH, L), jnp.float32),
            pltpu.VMEM_SHARED((n_pad, L), jnp.float32),
            pltpu.SemaphoreType.DMA((NBI,)),
            pltpu.SemaphoreType.DMA((NBI,)),
        ],
        compiler_params=_SC_PARAMS,
    )
    def deg(srcf_hbm, dacc_hbm, iring, ones_v, dacc_sh, isem, ssem):
        c = lax.axis_index("c")
        s = lax.axis_index("s")
        wid = c * NS + s

        _fill_f32(ones_v, CH, 1, 0.0)

        def zloop(j, _):
            pltpu.sync_copy(ones_v, dacc_sh.at[pl.ds(s * rpt + j * CH, CH)])
            return 0
        lax.fori_loop(0, rpt // CH, zloop, 0)
        _fill_f32(ones_v, CH, 1, 1.0)
        plsc.subcore_barrier()

        def istart(j):
            b = lax.rem(j, NBI)
            pltpu.async_copy(srcf_hbm.at[pl.ds((wid * nit + j) * CH, CH)],
                             iring.at[b], isem.at[b])

        def iwait(j):
            b = lax.rem(j, NBI)
            pltpu.make_async_copy(srcf_hbm.at[pl.ds((wid * nit + j) * CH, CH)],
                                  iring.at[b], isem.at[b]).wait()

        def sstart(i):
            b = lax.rem(i, NBI)
            pltpu.async_copy(ones_v, dacc_sh.at[iring.at[b]], ssem.at[b],
                             add=True)

        def swait(i):
            b = lax.rem(i, NBI)
            pltpu.make_async_copy(ones_v, dacc_sh.at[iring.at[b]],
                                  ssem.at[b]).wait()

        istart(0)
        istart(1)

        def chunk(i, _):
            jj = i + 2
            @pl.when(jj < nit)
            def _():
                @pl.when(jj >= NBI)
                def _():
                    swait(jj - NBI)
                istart(jj)
            iwait(i)
            sstart(i)
            return 0
        lax.fori_loop(0, nit, chunk, 0)

        def drain(t, _):
            swait(nit - NBI + t)
            return 0
        lax.fori_loop(0, NBI, drain, 0)
        plsc.subcore_barrier()

        def flush(j, _):
            r0 = s * rpt + j * FR
            pltpu.sync_copy(dacc_sh.at[pl.ds(r0, FR)], dacc_hbm.at[c, pl.ds(r0, FR)])
            return 0
        lax.fori_loop(0, rpt // FR, flush, 0)

    return deg


@functools.lru_cache(maxsize=None)
def _spmm_kernel(n_nodes, nit, d, dst_base):
    rpt, n_pad = _pad_rows(n_nodes)

    @functools.partial(
        pl.kernel,
        out_type=jax.ShapeDtypeStruct((NC, n_pad, d), jnp.float32),
        mesh=_mesh(),
        scratch_types=[
            pltpu.VMEM((NBI, CH), jnp.int32),
            pltpu.VMEM((NBI, CH), jnp.int32),
            pltpu.VMEM((NB, CH, d), jnp.float32),
            pltpu.VMEM_SHARED((n_pad, d), jnp.float32),
            pltpu.SemaphoreType.DMA((NBI,)),
            pltpu.SemaphoreType.DMA((NB,)),
            pltpu.SemaphoreType.DMA((NB,)),
        ],
        compiler_params=_SC_PARAMS,
    )
    def spmm(hw_hbm, ef_hbm, sout_hbm, isr_v, idr_v, rows_v,
             s_sh, isem, gsem, ssem):
        c = lax.axis_index("c")
        s = lax.axis_index("s")
        wid = c * NS + s

        def istart(j):
            b = lax.rem(j, NBI)
            off = (wid * nit + j) * CH
            pltpu.async_copy(ef_hbm.at[pl.ds(off, CH)], isr_v.at[b], isem.at[b])
            pltpu.async_copy(ef_hbm.at[pl.ds(dst_base + off, CH)], idr_v.at[b],
                             isem.at[b])

        def iwait(j):
            b = lax.rem(j, NBI)
            off = (wid * nit + j) * CH
            pltpu.make_async_copy(
                ef_hbm.at[pl.ds(off, CH)], isr_v.at[b], isem.at[b]).wait()
            pltpu.make_async_copy(
                ef_hbm.at[pl.ds(dst_base + off, CH)], idr_v.at[b],
                isem.at[b]).wait()

        def gstart(j):
            pltpu.async_copy(hw_hbm.at[isr_v.at[lax.rem(j, NBI)]],
                             rows_v.at[lax.rem(j, NB)], gsem.at[lax.rem(j, NB)])

        def gwait(j):
            pltpu.make_async_copy(hw_hbm.at[isr_v.at[lax.rem(j, NBI)]],
                                  rows_v.at[lax.rem(j, NB)],
                                  gsem.at[lax.rem(j, NB)]).wait()

        def sstart(i):
            pltpu.async_copy(rows_v.at[lax.rem(i, NB)],
                             s_sh.at[idr_v.at[lax.rem(i, NBI)]],
                             ssem.at[lax.rem(i, NB)], add=True)

        def swait(i):
            pltpu.make_async_copy(rows_v.at[lax.rem(i, NB)],
                                  s_sh.at[idr_v.at[lax.rem(i, NBI)]],
                                  ssem.at[lax.rem(i, NB)]).wait()

        istart(0)
        istart(1)
        iwait(0)
        gstart(0)

        def zfill(i, _):
            def col(j, __):
                rows_v[NB - 1, i, pl.ds(j * L, L)] = jnp.zeros((L,), jnp.float32)
                return 0
            return lax.fori_loop(0, d // L, col, 0)
        lax.fori_loop(0, CH, zfill, 0)

        def zloop(j, _):
            pltpu.sync_copy(rows_v.at[NB - 1], s_sh.at[pl.ds(s * rpt + j * CH, CH)])
            return 0
        lax.fori_loop(0, rpt // CH, zloop, 0)
        plsc.subcore_barrier()

        def chunk(i, _):
            j2 = i + 2
            @pl.when(j2 < nit)
            def _():
                istart(j2)
            j1 = i + 1
            @pl.when(j1 < nit)
            def _():
                @pl.when(j1 >= NB)
                def _():
                    swait(j1 - NB)
                iwait(j1)
                gstart(j1)
            gwait(i)
            sstart(i)
            return 0
        lax.fori_loop(0, nit, chunk, 0)

        def drain(t, _):
            swait(nit - NB + t)
            return 0
        lax.fori_loop(0, NB, drain, 0)
        plsc.subcore_barrier()

        def flush(j, _):
            r0 = s * rpt + j * FR
            pltpu.sync_copy(s_sh.at[pl.ds(r0, FR)], sout_hbm.at[c, pl.ds(r0, FR)])
            return 0
        lax.fori_loop(0, rpt // FR, flush, 0)

    return spmm


def _dinv(dacc_blk):
    deg = dacc_blk[0, :, 0:1] + dacc_blk[1, :, 0:1] + 1.0
    return lax.rsqrt(deg)


@functools.lru_cache(maxsize=None)
def _tc_mm(n, d):
    def body(x_ref, w_ref, b_ref, out_ref):
        hw = jnp.dot(x_ref[...], w_ref[...], preferred_element_type=jnp.float32)
        out_ref[...] = hw + b_ref[...]

    return pl.pallas_call(
        body,
        grid=(n // TC_R,),
        in_specs=[
            pl.BlockSpec((TC_R, d), lambda i: (i, 0)),
            pl.BlockSpec((d, d), lambda i: (0, 0)),
            pl.BlockSpec((1, d), lambda i: (0, 0)),
        ],
        out_specs=pl.BlockSpec((TC_R, d), lambda i: (i, 0)),
        out_shape=jax.ShapeDtypeStruct((n, d), jnp.float32),
    )


@functools.lru_cache(maxsize=None)
def _tc_scale(n, d):
    def body(hw_ref, dacc_ref, out_ref):
        out_ref[...] = hw_ref[...] * _dinv(dacc_ref[...])

    return pl.pallas_call(
        body,
        grid=(n // TC_R,),
        in_specs=[
            pl.BlockSpec((TC_R, d), lambda i: (i, 0)),
            pl.BlockSpec((NC, TC_R, L), lambda i: (0, i, 0)),
        ],
        out_specs=pl.BlockSpec((TC_R, d), lambda i: (i, 0)),
        out_shape=jax.ShapeDtypeStruct((n, d), jnp.float32),
    )


@functools.lru_cache(maxsize=None)
def _tc_mid(n, d):
    def body(dacc_ref, s_ref, hwp_ref, w_ref, b_ref, out_ref):
        dinv = _dinv(dacc_ref[...])
        sm = s_ref[0] + s_ref[1] + hwp_ref[...]
        h = jnp.maximum(dinv * sm, 0.0)
        hw = jnp.dot(h, w_ref[...], preferred_element_type=jnp.float32)
        out_ref[...] = (hw + b_ref[...]) * dinv

    return pl.pallas_call(
        body,
        grid=(n // TC_R,),
        in_specs=[
            pl.BlockSpec((NC, TC_R, L), lambda i: (0, i, 0)),
            pl.BlockSpec((NC, TC_R, d), lambda i: (0, i, 0)),
            pl.BlockSpec((TC_R, d), lambda i: (i, 0)),
            pl.BlockSpec((d, d), lambda i: (0, 0)),
            pl.BlockSpec((1, d), lambda i: (0, 0)),
        ],
        out_specs=pl.BlockSpec((TC_R, d), lambda i: (i, 0)),
        out_shape=jax.ShapeDtypeStruct((n, d), jnp.float32),
    )


@functools.lru_cache(maxsize=None)
def _tc_last(n, d):
    def body(dacc_ref, s_ref, hwp_ref, out_ref):
        dinv = _dinv(dacc_ref[...])
        sm = s_ref[0] + s_ref[1] + hwp_ref[...]
        out_ref[...] = jnp.maximum(dinv * sm, 0.0)

    return pl.pallas_call(
        body,
        grid=(n // TC_R,),
        in_specs=[
            pl.BlockSpec((NC, TC_R, L), lambda i: (0, i, 0)),
            pl.BlockSpec((NC, TC_R, d), lambda i: (0, i, 0)),
            pl.BlockSpec((TC_R, d), lambda i: (i, 0)),
        ],
        out_specs=pl.BlockSpec((TC_R, d), lambda i: (i, 0)),
        out_shape=jax.ShapeDtypeStruct((n, d), jnp.float32),
    )


def kernel(edge_index, X, W1, b1, W2, b2):
    n, d = X.shape
    e = edge_index.shape[1]
    assert n % TC_R == 0 and d % L == 0

    _, n_pad = _pad_rows(n)
    block = NW * CH
    nit = max(8, -(-e // block))
    e_pad = NW * nit * CH
    npadlen = e_pad - e

    b1r = b1.reshape(1, d)
    b2r = b2.reshape(1, d)
    if npadlen == 0:
        ef = edge_index.reshape(2 * e)
        ef_deg = ef
        dst_base = e
    else:
        src = edge_index[0]
        dst = edge_index[1]
        pad_last = jnp.full((npadlen,), n_pad - 1, jnp.int32)
        ef = jnp.concatenate(
            [src, jnp.zeros((npadlen,), jnp.int32), dst, pad_last])
        ef_deg = jnp.concatenate([src, pad_last])
        dst_base = e_pad

    dacc = _deg_kernel(n, nit, ef_deg.shape[0])(ef_deg)
    hw1 = _tc_mm(n, d)(X, W1, b1r)
    hw1p = _tc_scale(n, d)(hw1, dacc)
    s1 = _spmm_kernel(n, nit, d, dst_base)(hw1p, ef)
    hw2p = _tc_mid(n, d)(dacc, s1, hw1p, W2, b2r)
    s2 = _spmm_kernel(n, nit, d, dst_base)(hw2p, ef)
    return _tc_last(n, d)(dacc, s2, hw2p)
